# baseline (speedup 1.0000x reference)
import jax
import jax.numpy as jnp
from jax.experimental import pallas as pl

_B = 4096
_L = 200
_D = 32
_BLK = 64


def _body(w_ref, out_ref):
    s = w_ref[0, 0, 0]
    out_ref[...] = jnp.full((_BLK, _L, _D), s, dtype=jnp.float32)


def kernel(labels, weight):
    wslice = jax.lax.slice(weight, (1, 0), (1 + _L, _D)).reshape(1, _L, _D)
    out = pl.pallas_call(
        _body,
        grid=(_B // _BLK,),
        in_specs=[
            pl.BlockSpec((1, _L, _D), lambda i: (0, 0, 0)),
        ],
        out_specs=pl.BlockSpec((_BLK, _L, _D), lambda i: (i, 0, 0)),
        out_shape=jax.ShapeDtypeStruct((_B, _L, _D), jnp.float32),
    )(wslice)
    return out


# speedup vs baseline: 8.4530x; 8.4530x over previous
import jax
import jax.numpy as jnp
from jax.experimental import pallas as pl

_B = 4096
_L = 200
_D = 32
_BLK = 512


def _body(labelsT_ref, ewT_ref, out_ref):
    m = (labelsT_ref[...] != 0).astype(jnp.float32)
    out_ref[...] = jax.lax.dot(ewT_ref[...], m,
                               preferred_element_type=jnp.float32)


def kernel(labels, weight):
    wflat = jax.lax.slice(weight, (1, 0), (1 + _L, _D)).reshape(_L * _D)
    row = jnp.arange(_L * _D, dtype=jnp.int32) // _D
    onehot = (row[:, None] == jnp.arange(_L, dtype=jnp.int32)[None, :])
    ewT = onehot.astype(jnp.float32) * wflat[:, None]
    labelsT = labels.T
    outT = pl.pallas_call(
        _body,
        grid=(_B // _BLK,),
        in_specs=[
            pl.BlockSpec((_L, _BLK), lambda i: (0, i)),
            pl.BlockSpec((_L * _D, _L), lambda i: (0, 0)),
        ],
        out_specs=pl.BlockSpec((_L * _D, _BLK), lambda i: (0, i)),
        out_shape=jax.ShapeDtypeStruct((_L * _D, _B), jnp.float32),
    )(labelsT, ewT)
    return outT.reshape(_L, _D, _B).transpose(2, 0, 1)
